# baseline (speedup 1.0000x reference)
import functools

import jax
import jax.numpy as jnp
from jax import lax
from jax.experimental import pallas as pl
from jax.experimental.pallas import tpu as pltpu
from jax.experimental.pallas import tpu_sc as plsc

BATCH = 16384
HIST = 50
DIM = 64
NC = 2
NS = 16
NW = NC * NS
BW = BATCH // NW
CB = 256
NSUB = BW // CB
NT = HIST * NSUB

_mesh = plsc.VectorSubcoreMesh(core_axis_name="c", subcore_axis_name="s")


@functools.partial(
    pl.kernel,
    out_type=jax.ShapeDtypeStruct((HIST, DIM, BATCH), jnp.float32),
    mesh=_mesh,
    scratch_types=[
        pltpu.VMEM((HIST, BW), jnp.int32),
        pltpu.VMEM((CB, DIM), jnp.float32),
        pltpu.VMEM((CB, DIM), jnp.float32),
        pltpu.VMEM((DIM, CB), jnp.float32),
        pltpu.VMEM((DIM, CB), jnp.float32),
        pltpu.SemaphoreType.DMA,
        pltpu.SemaphoreType.DMA,
        pltpu.SemaphoreType.DMA,
        pltpu.SemaphoreType.DMA,
        pltpu.SemaphoreType.DMA,
    ],
    compiler_params=pltpu.CompilerParams(
        use_tc_tiling_on_sc=False, needs_layout_passes=False
    ),
)
def _gather_kernel(table_hbm, idxt_hbm, out_hbm,
                   idx_all, rows0, rows1, rt0, rt1,
                   s_idx, sg0, sg1, sw0, sw1):
    wid = lax.axis_index("s") * NC + lax.axis_index("c")
    boff = wid * BW
    rows = (rows0, rows1)
    rt = (rt0, rt1)
    s_g = (sg0, sg1)
    s_w = (sw0, sw1)
    iota = lax.iota(jnp.int32, 16)

    def fire_gather(h, sub, b):
        idx_slice = idx_all.at[h, pl.ds(sub * CB, CB)]
        pltpu.async_copy(table_hbm.at[idx_slice], rows[b], s_g[b])

    def transpose(b):
        def rblk(r, carry):
            r0 = r * 16
            rids = r0 + iota
            for c0 in (0, 16, 32, 48):
                for j in range(16):
                    cids = c0 + ((iota + j) & 15)
                    v = plsc.load_gather(rows[b], [rids, cids])
                    plsc.store_scatter(rt[b], [cids, rids], v)
            return carry
        lax.fori_loop(0, CB // 16, rblk, 0)

    def step(h, b, *, first, fire_next):
        if fire_next:
            fire_gather(h + b, 1 - b, 1 - b)
        pltpu.make_async_copy(
            table_hbm.at[idx_all.at[h, pl.ds(b * CB, CB)]], rows[b], s_g[b]
        ).wait()
        if not first:
            pltpu.make_async_copy(
                rt[b], out_hbm.at[h, :, pl.ds(boff + b * CB, CB)], s_w[b]
            ).wait()
        transpose(b)
        pltpu.async_copy(
            rt[b], out_hbm.at[h, :, pl.ds(boff + b * CB, CB)], s_w[b]
        )

    pltpu.async_copy(idxt_hbm.at[:, pl.ds(boff, BW)], idx_all, s_idx).wait()
    fire_gather(0, 0, 0)

    step(0, 0, first=True, fire_next=True)
    step(0, 1, first=True, fire_next=True)

    def body(h, carry):
        step(h, 0, first=False, fire_next=True)
        step(h, 1, first=False, fire_next=True)
        return carry

    lax.fori_loop(1, HIST - 1, body, 0)

    step(HIST - 1, 0, first=False, fire_next=True)
    step(HIST - 1, 1, first=False, fire_next=False)

    h_last = HIST - 1
    pltpu.make_async_copy(
        rt0, out_hbm.at[h_last, :, pl.ds(boff, CB)], sw0
    ).wait()
    pltpu.make_async_copy(
        rt1, out_hbm.at[h_last, :, pl.ds(boff + CB, CB)], sw1
    ).wait()


def kernel(ivectors, data):
    idx_t = data.T.astype(jnp.int32)
    out_p = _gather_kernel(ivectors, idx_t)
    return out_p.transpose(2, 0, 1)


# speedup vs baseline: 1.3849x; 1.3849x over previous
import functools

import jax
import jax.numpy as jnp
from jax import lax
from jax.experimental import pallas as pl
from jax.experimental.pallas import tpu as pltpu
from jax.experimental.pallas import tpu_sc as plsc

BATCH = 16384
HIST = 50
DIM = 64
NC = 2
NS = 16
NW = NC * NS
BW = BATCH // NW
CB = 256
NSUB = BW // CB
NT = HIST * NSUB

_mesh = plsc.VectorSubcoreMesh(core_axis_name="c", subcore_axis_name="s")


@functools.partial(
    pl.kernel,
    out_type=jax.ShapeDtypeStruct((HIST, DIM, BATCH), jnp.float32),
    mesh=_mesh,
    scratch_types=[
        pltpu.VMEM((HIST, BW), jnp.int32),
        pltpu.VMEM((CB, DIM), jnp.float32),
        pltpu.VMEM((CB, DIM), jnp.float32),
        pltpu.VMEM((DIM, CB), jnp.float32),
        pltpu.VMEM((DIM, CB), jnp.float32),
        pltpu.SemaphoreType.DMA,
        pltpu.SemaphoreType.DMA,
        pltpu.SemaphoreType.DMA,
        pltpu.SemaphoreType.DMA,
        pltpu.SemaphoreType.DMA,
    ],
    compiler_params=pltpu.CompilerParams(
        use_tc_tiling_on_sc=False, needs_layout_passes=False
    ),
)
def _gather_kernel(table_hbm, idxt_hbm, out_hbm,
                   idx_all, rows0, rows1, rt0, rt1,
                   s_idx, sg0, sg1, sw0, sw1):
    wid = lax.axis_index("s") * NC + lax.axis_index("c")
    boff = wid * BW
    rows = (rows0, rows1)
    rt = (rt0, rt1)
    s_g = (sg0, sg1)
    s_w = (sw0, sw1)
    iota = lax.iota(jnp.int32, 16)

    def fire_gather(h, sub, b):
        half = CB // 2
        for p in range(2):
            idx_slice = idx_all.at[h, pl.ds(sub * CB + p * half, half)]
            pltpu.async_copy(
                table_hbm.at[idx_slice], rows[b].at[pl.ds(p * half, half)], s_g[b]
            )

    def wait_gather(h, sub, b):
        half = CB // 2
        for p in range(2):
            idx_slice = idx_all.at[h, pl.ds(sub * CB + p * half, half)]
            pltpu.make_async_copy(
                table_hbm.at[idx_slice], rows[b].at[pl.ds(p * half, half)], s_g[b]
            ).wait()

    def transpose(b):
        def rblk(r, carry):
            r0 = r * 16
            rids = r0 + iota
            for c0 in (0, 16, 32, 48):
                cid_list = [c0 + ((iota + j) & 15) for j in range(16)]
                vs = [plsc.load_gather(rows[b], [rids, cids]) for cids in cid_list]
                for cids, v in zip(cid_list, vs):
                    plsc.store_scatter(rt[b], [cids, rids], v)
            return carry
        lax.fori_loop(0, CB // 16, rblk, 0)

    def step(h, b, *, first, fire_next):
        if fire_next:
            fire_gather(h + b, 1 - b, 1 - b)
        wait_gather(h, b, b)
        if not first:
            pltpu.make_async_copy(
                rt[b], out_hbm.at[h, :, pl.ds(boff + b * CB, CB)], s_w[b]
            ).wait()
        transpose(b)
        pltpu.async_copy(
            rt[b], out_hbm.at[h, :, pl.ds(boff + b * CB, CB)], s_w[b]
        )

    pltpu.async_copy(idxt_hbm.at[:, pl.ds(boff, BW)], idx_all, s_idx).wait()
    fire_gather(0, 0, 0)

    step(0, 0, first=True, fire_next=True)
    step(0, 1, first=True, fire_next=True)

    def body(h, carry):
        step(h, 0, first=False, fire_next=True)
        step(h, 1, first=False, fire_next=True)
        return carry

    lax.fori_loop(1, HIST - 1, body, 0)

    step(HIST - 1, 0, first=False, fire_next=True)
    step(HIST - 1, 1, first=False, fire_next=False)

    h_last = HIST - 1
    pltpu.make_async_copy(
        rt0, out_hbm.at[h_last, :, pl.ds(boff, CB)], sw0
    ).wait()
    pltpu.make_async_copy(
        rt1, out_hbm.at[h_last, :, pl.ds(boff + CB, CB)], sw1
    ).wait()


def kernel(ivectors, data):
    idx_t = data.T.astype(jnp.int32)
    out_p = _gather_kernel(ivectors, idx_t)
    return out_p.transpose(2, 0, 1)
